# baseline (speedup 1.0000x reference)
import jax
import jax.numpy as jnp
from jax.experimental import pallas as pl

_T = 2048


def _lrelu(x):
    return jnp.where(x >= 0, x, 0.01 * x)


def _first_max(y, k):
    maxv = jnp.max(y, axis=1, keepdims=True)
    li = jax.lax.broadcasted_iota(jnp.int32, y.shape, 1)
    return jnp.min(jnp.where(y == maxv, li, k), axis=1, keepdims=True)


def _select_expert(a, mask, group):
    a = jnp.where(mask, a, 0.0)
    while a.shape[1] > group:
        h = a.shape[1] // 2
        a = a[:, :h] + a[:, h:]
    return a


def _moe_kernel(x_ref, w01_ref, b1_ref, w2_ref, b2_ref, w3_ref, b3_ref,
                b21_ref, w22_ref, b22_ref, w23_ref, b23_ref,
                w31_ref, b31_ref, w32_ref, b32_ref, w33_ref, b33_ref,
                id16_ref, id21_ref, id23_ref, id2048_ref, id33_ref,
                xr_ref, mask_ref):
    f32 = jnp.float32
    bf16 = jnp.bfloat16
    x = jnp.transpose(x_ref[0, :, 0, :])
    a0 = jnp.dot(x, w01_ref[...], preferred_element_type=f32)
    h = _lrelu(a0[:, :128] + b1_ref[...])
    h = _lrelu(jnp.dot(h, w2_ref[...], preferred_element_type=f32) + b2_ref[...])
    y3 = jnp.dot(h, w3_ref[...], preferred_element_type=f32) + b3_ref[...]
    mask_ref[...] = _lrelu(y3[:, 16:17])
    inds1 = _first_max(y3[:, :16], 16)
    inds1f = inds1.astype(f32)
    t = inds1f.shape[0]
    bc1 = jnp.broadcast_to(inds1f, (t, 512))
    m1_512 = id21_ref[...] == bc1
    m1_256 = id23_ref[...] == bc1[:, :256]
    a = a0[:, 128:] + b21_ref[...]
    g = _lrelu(_select_expert(a, m1_512, 32))
    a = jnp.dot(g, w22_ref[...], preferred_element_type=f32) + b22_ref[...]
    g = _lrelu(_select_expert(a, m1_512, 32))
    a = jnp.dot(g, w23_ref[...], preferred_element_type=f32) + b23_ref[...]
    x2 = _select_expert(a, m1_256, 16)
    inds2 = _first_max(x2, 16)
    inds2f = inds2.astype(f32)
    inds12 = inds1 * 16 + inds2
    inds12f = inds12.astype(f32)
    bc2 = jnp.broadcast_to(inds2f, (t, 512))
    m2_512 = id21_ref[...] == bc2
    m2_256 = id23_ref[...] == bc2[:, :256]
    m2_16 = id16_ref[...] == bc2[:, :16]
    onehot = jnp.where(id33_ref[...] == jnp.broadcast_to(inds12f, (t, 256)), 1.0, 0.0)
    xz = jnp.where(id2048_ref[...] == jnp.broadcast_to(inds1.astype(bf16), (t, 2048)),
                   jnp.tile(x.astype(bf16), (1, 16)), 0.0)
    a = jnp.dot(xz, w31_ref[...], preferred_element_type=f32)
    g = _select_expert(a, m2_512, 32)
    g = _lrelu(g + jnp.dot(onehot, b31_ref[...], preferred_element_type=f32))
    z = jnp.where(m1_512, jnp.tile(g, (1, 16)), 0.0)
    a = jnp.dot(z, w32_ref[...], preferred_element_type=f32)
    g = _select_expert(a, m2_256, 16)
    g = _lrelu(g + jnp.dot(onehot, b32_ref[...], preferred_element_type=f32))
    z = jnp.where(m1_256, jnp.tile(g, (1, 16)), 0.0)
    a = jnp.dot(z, w33_ref[...], preferred_element_type=f32)
    r = jnp.sum(jnp.where(m2_16, a, 0.0), axis=1, keepdims=True)
    r = r + jnp.sum(onehot * b33_ref[...], axis=1, keepdims=True)
    xr_ref[...] = (inds12f + r) * (1.0 / 256.0)


def kernel(x_in, c1_1_w, c1_1_b, c1_2_w, c1_2_b, c1_3_w, c1_3_b,
           c2_1_w, c2_1_b, c2_2_w, c2_2_b, c2_3_w, c2_3_b,
           r1_1_w, r1_1_b, r1_2_w, r1_2_b, r1_3_w, r1_3_b):
    B, C, H, W = x_in.shape
    n = B * H * W
    f32 = jnp.float32
    bf16 = jnp.bfloat16

    def _eid(width, group, dtype):
        return (jnp.arange(width, dtype=jnp.int32) // group).astype(dtype).reshape(1, width)

    params = (
        jnp.concatenate([c1_1_w.T, c2_1_w.transpose(1, 0, 2).reshape(C, -1)], axis=1),
        c1_1_b.reshape(1, -1),
        c1_2_w.T, c1_2_b.reshape(1, -1),
        jnp.pad(c1_3_w.T, ((0, 0), (0, 15))), jnp.pad(c1_3_b, (0, 15)).reshape(1, -1),
        c2_1_b.reshape(1, -1),
        c2_2_w.transpose(1, 0, 2).reshape(32, -1), c2_2_b.reshape(1, -1),
        c2_3_w.transpose(1, 0, 2).reshape(32, -1), c2_3_b.reshape(1, -1),
        r1_1_w.reshape(16, 16, C, 32).transpose(0, 2, 1, 3).reshape(16 * C, 512).astype(bf16),
        r1_1_b,
        r1_2_w.reshape(16, 16, 32, 16).transpose(0, 2, 1, 3).reshape(512, 256),
        r1_2_b,
        r1_3_w.reshape(16, 16, 16).transpose(0, 2, 1).reshape(256, 16),
        r1_3_b.reshape(1, -1),
        _eid(16, 1, f32), _eid(512, 32, f32), _eid(256, 16, f32),
        _eid(16 * C, C, bf16), _eid(256, 1, f32),
    )

    def _const(shape):
        return pl.BlockSpec(shape, lambda i: (0, 0))

    tb = min(_T, W)
    bw = W // tb
    in_specs = [pl.BlockSpec((1, C, 1, tb), lambda i: (i // bw, 0, 0, i % bw))]
    in_specs += [_const(p.shape) for p in params]
    out_specs = [pl.BlockSpec((tb, 1), lambda i: (i, 0)),
                 pl.BlockSpec((tb, 1), lambda i: (i, 0))]
    out_shape = (jax.ShapeDtypeStruct((n, 1), f32),
                 jax.ShapeDtypeStruct((n, 1), f32))

    xr, mask = pl.pallas_call(
        _moe_kernel,
        grid=(n // tb,),
        in_specs=in_specs,
        out_specs=out_specs,
        out_shape=out_shape,
    )(x_in, *params)
    return xr.reshape(B, 1, H, W), mask.reshape(B, 1, H, W)


# speedup vs baseline: 1.0405x; 1.0405x over previous
import jax
import jax.numpy as jnp
from jax.experimental import pallas as pl

_T = 1024


def _lrelu(x):
    return jnp.where(x >= 0, x, 0.01 * x)


def _first_max(y, k):
    maxv = jnp.max(y, axis=1, keepdims=True)
    li = jax.lax.broadcasted_iota(jnp.int32, y.shape, 1)
    return jnp.min(jnp.where(y == maxv, li, k), axis=1, keepdims=True)


def _select_expert(a, mask, group):
    a = jnp.where(mask, a, 0.0)
    while a.shape[1] > group:
        h = a.shape[1] // 2
        a = a[:, :h] + a[:, h:]
    return a


def _moe_kernel(x_ref, w01_ref, b1_ref, w2_ref, b2_ref, w3_ref, b3_ref,
                b21_ref, w22_ref, b22_ref, w23_ref, b23_ref,
                w31_ref, b31_ref, w32_ref, b32_ref, w33_ref, b33_ref,
                id16_ref, id21_ref, id23_ref, id2048_ref, id33_ref,
                xr_ref, mask_ref):
    f32 = jnp.float32
    bf16 = jnp.bfloat16
    x = jnp.transpose(x_ref[0, :, 0, :])
    a0 = jnp.dot(x, w01_ref[...], preferred_element_type=f32)
    h = _lrelu(a0[:, :128] + b1_ref[...])
    h = _lrelu(jnp.dot(h, w2_ref[...], preferred_element_type=f32) + b2_ref[...])
    y3 = jnp.dot(h, w3_ref[...], preferred_element_type=f32) + b3_ref[...]
    mask_ref[...] = _lrelu(y3[:, 16:17])
    inds1 = _first_max(y3[:, :16], 16)
    inds1f = inds1.astype(f32)
    t = inds1f.shape[0]
    bc1 = jnp.broadcast_to(inds1f, (t, 512))
    m1_512 = id21_ref[...] == bc1
    m1_256 = id23_ref[...] == bc1[:, :256]
    a = a0[:, 128:] + b21_ref[...]
    g = _lrelu(_select_expert(a, m1_512, 32))
    a = jnp.dot(g, w22_ref[...], preferred_element_type=f32) + b22_ref[...]
    g = _lrelu(_select_expert(a, m1_512, 32))
    a = jnp.dot(g, w23_ref[...], preferred_element_type=f32) + b23_ref[...]
    x2 = _select_expert(a, m1_256, 16)
    inds2 = _first_max(x2, 16)
    inds2f = inds2.astype(f32)
    inds12 = inds1 * 16 + inds2
    inds12f = inds12.astype(f32)
    bc2 = jnp.broadcast_to(inds2f, (t, 512))
    m2_512 = id21_ref[...] == bc2
    m2_256 = id23_ref[...] == bc2[:, :256]
    m2_16 = id16_ref[...] == bc2[:, :16]
    onehot = jnp.where(id33_ref[...] == jnp.broadcast_to(inds12f, (t, 256)), 1.0, 0.0)
    xz = jnp.where(id2048_ref[...] == jnp.broadcast_to(inds1.astype(bf16), (t, 2048)),
                   jnp.tile(x.astype(bf16), (1, 16)), 0.0)
    a = jnp.dot(xz, w31_ref[...], preferred_element_type=f32)
    g = _select_expert(a, m2_512, 32)
    g = _lrelu(g + jnp.dot(onehot, b31_ref[...], preferred_element_type=f32))
    z = jnp.where(m1_512, jnp.tile(g, (1, 16)), 0.0)
    a = jnp.dot(z, w32_ref[...], preferred_element_type=f32)
    g = _select_expert(a, m2_256, 16)
    g = _lrelu(g + jnp.dot(onehot, b32_ref[...], preferred_element_type=f32))
    z = jnp.where(m1_256, jnp.tile(g, (1, 16)), 0.0)
    a = jnp.dot(z, w33_ref[...], preferred_element_type=f32)
    r = jnp.sum(jnp.where(m2_16, a, 0.0), axis=1, keepdims=True)
    r = r + jnp.sum(onehot * b33_ref[...], axis=1, keepdims=True)
    xr_ref[...] = (inds12f + r) * (1.0 / 256.0)


def kernel(x_in, c1_1_w, c1_1_b, c1_2_w, c1_2_b, c1_3_w, c1_3_b,
           c2_1_w, c2_1_b, c2_2_w, c2_2_b, c2_3_w, c2_3_b,
           r1_1_w, r1_1_b, r1_2_w, r1_2_b, r1_3_w, r1_3_b):
    B, C, H, W = x_in.shape
    n = B * H * W
    f32 = jnp.float32
    bf16 = jnp.bfloat16

    def _eid(width, group, dtype):
        return (jnp.arange(width, dtype=jnp.int32) // group).astype(dtype).reshape(1, width)

    params = (
        jnp.concatenate([c1_1_w.T, c2_1_w.transpose(1, 0, 2).reshape(C, -1)], axis=1),
        c1_1_b.reshape(1, -1),
        c1_2_w.T, c1_2_b.reshape(1, -1),
        jnp.pad(c1_3_w.T, ((0, 0), (0, 15))), jnp.pad(c1_3_b, (0, 15)).reshape(1, -1),
        c2_1_b.reshape(1, -1),
        c2_2_w.transpose(1, 0, 2).reshape(32, -1), c2_2_b.reshape(1, -1),
        c2_3_w.transpose(1, 0, 2).reshape(32, -1), c2_3_b.reshape(1, -1),
        r1_1_w.reshape(16, 16, C, 32).transpose(0, 2, 1, 3).reshape(16 * C, 512).astype(bf16),
        r1_1_b,
        r1_2_w.reshape(16, 16, 32, 16).transpose(0, 2, 1, 3).reshape(512, 256),
        r1_2_b,
        r1_3_w.reshape(16, 16, 16).transpose(0, 2, 1).reshape(256, 16),
        r1_3_b.reshape(1, -1),
        _eid(16, 1, f32), _eid(512, 32, f32), _eid(256, 16, f32),
        _eid(16 * C, C, bf16), _eid(256, 1, f32),
    )

    def _const(shape):
        return pl.BlockSpec(shape, lambda i: (0, 0))

    tb = min(_T, W)
    bw = W // tb
    in_specs = [pl.BlockSpec((1, C, 1, tb), lambda i: (i // bw, 0, 0, i % bw))]
    in_specs += [_const(p.shape) for p in params]
    out_specs = [pl.BlockSpec((tb, 1), lambda i: (i, 0)),
                 pl.BlockSpec((tb, 1), lambda i: (i, 0))]
    out_shape = (jax.ShapeDtypeStruct((n, 1), f32),
                 jax.ShapeDtypeStruct((n, 1), f32))

    xr, mask = pl.pallas_call(
        _moe_kernel,
        grid=(n // tb,),
        in_specs=in_specs,
        out_specs=out_specs,
        out_shape=out_shape,
    )(x_in, *params)
    return xr.reshape(B, 1, H, W), mask.reshape(B, 1, H, W)
